# baseline (speedup 1.0000x reference)
import functools

import jax
import jax.numpy as jnp
from jax import lax
from jax.experimental import pallas as pl
from jax.experimental.pallas import tpu as pltpu
from jax.experimental.pallas import tpu_sc as plsc

_NUM_CORES = 2
_NUM_SUBCORES = 16
_NW = _NUM_CORES * _NUM_SUBCORES

_G = 40
_K = 4


def _make_gather_kernel(batch: int, seq: int, d: int):
    rows_per_w = batch // _NW
    chunks_per_row = seq // _G
    n_chunks = rows_per_w * chunks_per_row
    n_groups = n_chunks // _K
    assert batch % _NW == 0 and seq % _G == 0
    assert n_chunks % _K == 0 and n_groups % 2 == 0 and n_groups >= 4
    mesh = plsc.VectorSubcoreMesh(core_axis_name="c", subcore_axis_name="s")

    @functools.partial(
        pl.kernel,
        mesh=mesh,
        out_type=jax.ShapeDtypeStruct((batch, seq, 128), jnp.float32),
        scratch_types=[
            pltpu.VMEM((rows_per_w, seq), jnp.int32),
            pltpu.VMEM((2 * _K, _G, 128), jnp.float32),
            pltpu.SemaphoreType.DMA((2 * _K,)),
            pltpu.SemaphoreType.DMA((2 * _K,)),
        ],
        compiler_params=pltpu.CompilerParams(use_tc_tiling_on_sc=False),
    )
    def k(idx_hbm, table_hbm, out_hbm, idx_v, rows_v, gsem, osem):
        wid = lax.axis_index("s") * _NUM_CORES + lax.axis_index("c")
        row0 = wid * rows_per_w
        pltpu.sync_copy(idx_hbm.at[pl.ds(row0, rows_per_w)], idx_v)

        def chunk_coords(c):
            r = c // chunks_per_row
            j = c % chunks_per_row
            return r, j * _G

        def fire_gather(i, s):
            for b in range(_K):
                r, col = chunk_coords(i * _K + b)
                pltpu.async_copy(
                    table_hbm.at[idx_v.at[r, pl.ds(col, _G)]],
                    rows_v.at[s * _K + b],
                    gsem.at[s * _K + b],
                )

        def drain_gather(i, s):
            for b in range(_K):
                r, col = chunk_coords(i * _K + b)
                pltpu.make_async_copy(
                    table_hbm.at[idx_v.at[r, pl.ds(col, _G)]],
                    rows_v.at[s * _K + b],
                    gsem.at[s * _K + b],
                ).wait()

        def fire_out(i, s):
            for b in range(_K):
                r, col = chunk_coords(i * _K + b)
                pltpu.async_copy(
                    rows_v.at[s * _K + b, slice(None), pl.ds(0, d)],
                    out_hbm.at[row0 + r, pl.ds(col, _G), pl.ds(0, d)],
                    osem.at[s * _K + b],
                )

        def drain_out(i, s):
            for b in range(_K):
                r, col = chunk_coords(i * _K + b)
                pltpu.make_async_copy(
                    rows_v.at[s * _K + b, slice(None), pl.ds(0, d)],
                    out_hbm.at[row0 + r, pl.ds(col, _G), pl.ds(0, d)],
                    osem.at[s * _K + b],
                ).wait()

        fire_gather(0, 0)
        fire_gather(1, 1)
        drain_gather(0, 0)
        fire_out(0, 0)

        def steady(i2, carry):
            for g in range(2):
                i = 2 * i2 + 1 + g
                s = (1 + g) % 2
                s1 = 1 - s
                drain_out(i - 1, s1)
                fire_gather(i + 1, s1)
                drain_gather(i, s)
                fire_out(i, s)
            return carry

        lax.fori_loop(0, (n_groups - 2) // 2, steady, 0)

        last = n_groups - 1
        s_last = last % 2
        drain_gather(last, s_last)
        fire_out(last, s_last)
        drain_out(last - 1, 1 - s_last)
        drain_out(last, s_last)

    return k


def kernel(x, weight):
    batch, seq = x.shape
    d = weight.shape[1]
    wp = jnp.pad(weight, ((0, 0), (0, 128 - d)))
    return _make_gather_kernel(batch, seq, d)(x, wp)[..., :d]


# speedup vs baseline: 1.0130x; 1.0130x over previous
import functools

import jax
import jax.numpy as jnp
from jax import lax
from jax.experimental import pallas as pl
from jax.experimental.pallas import tpu as pltpu
from jax.experimental.pallas import tpu_sc as plsc

_NUM_CORES = 2
_NUM_SUBCORES = 16
_NW = _NUM_CORES * _NUM_SUBCORES

_G = 40
_K = 4


def _make_gather_kernel(batch: int, seq: int, d: int):
    rows_per_w = batch // _NW
    chunks_per_row = seq // _G
    n_chunks = rows_per_w * chunks_per_row
    n_groups = n_chunks // _K
    assert batch % _NW == 0 and seq % _G == 0
    assert n_chunks % _K == 0 and n_groups % 2 == 0 and n_groups >= 4
    mesh = plsc.VectorSubcoreMesh(core_axis_name="c", subcore_axis_name="s")

    @functools.partial(
        pl.kernel,
        mesh=mesh,
        out_type=jax.ShapeDtypeStruct((batch, seq, 128), jnp.float32),
        scratch_types=[
            pltpu.VMEM((rows_per_w, seq), jnp.int32),
            pltpu.VMEM((2 * _K, _G, d), jnp.float32),
            pltpu.SemaphoreType.DMA((2 * _K,)),
            pltpu.SemaphoreType.DMA((2 * _K,)),
        ],
        compiler_params=pltpu.CompilerParams(use_tc_tiling_on_sc=False),
    )
    def k(idx_hbm, table_hbm, out_hbm, idx_v, rows_v, gsem, osem):
        wid = lax.axis_index("s") * _NUM_CORES + lax.axis_index("c")
        row0 = wid * rows_per_w
        pltpu.sync_copy(idx_hbm.at[pl.ds(row0, rows_per_w)], idx_v)

        def chunk_coords(c):
            r = c // chunks_per_row
            j = c % chunks_per_row
            return r, j * _G

        def fire_gather(i, s):
            for b in range(_K):
                r, col = chunk_coords(i * _K + b)
                pltpu.async_copy(
                    table_hbm.at[idx_v.at[r, pl.ds(col, _G)]],
                    rows_v.at[s * _K + b],
                    gsem.at[s * _K + b],
                )

        def drain_gather(i, s):
            for b in range(_K):
                r, col = chunk_coords(i * _K + b)
                pltpu.make_async_copy(
                    table_hbm.at[idx_v.at[r, pl.ds(col, _G)]],
                    rows_v.at[s * _K + b],
                    gsem.at[s * _K + b],
                ).wait()

        def fire_out(i, s):
            for b in range(_K):
                r, col = chunk_coords(i * _K + b)
                pltpu.async_copy(
                    rows_v.at[s * _K + b],
                    out_hbm.at[row0 + r, pl.ds(col, _G), pl.ds(0, d)],
                    osem.at[s * _K + b],
                )

        def drain_out(i, s):
            for b in range(_K):
                r, col = chunk_coords(i * _K + b)
                pltpu.make_async_copy(
                    rows_v.at[s * _K + b],
                    out_hbm.at[row0 + r, pl.ds(col, _G), pl.ds(0, d)],
                    osem.at[s * _K + b],
                ).wait()

        fire_gather(0, 0)
        fire_gather(1, 1)
        drain_gather(0, 0)
        fire_out(0, 0)

        def steady(i2, carry):
            for g in range(2):
                i = 2 * i2 + 1 + g
                s = (1 + g) % 2
                s1 = 1 - s
                drain_out(i - 1, s1)
                fire_gather(i + 1, s1)
                drain_gather(i, s)
                fire_out(i, s)
            return carry

        lax.fori_loop(0, (n_groups - 2) // 2, steady, 0)

        last = n_groups - 1
        s_last = last % 2
        drain_gather(last, s_last)
        fire_out(last, s_last)
        drain_out(last - 1, 1 - s_last)
        drain_out(last, s_last)

    return k


def kernel(x, weight):
    batch, seq = x.shape
    d = weight.shape[1]
    return _make_gather_kernel(batch, seq, d)(x, weight)[..., :d]
